# Initial kernel scaffold; baseline (speedup 1.0000x reference)
import jax
import jax.numpy as jnp
from jax.experimental import pallas as pl


def kernel(x, edge_index, weights, biases):
    raise NotImplementedError("write your pallas kernel here")


# speedup vs baseline: 8.6722x; 8.6722x over previous
import functools

import jax
import jax.numpy as jnp
from jax import lax
from jax.experimental import pallas as pl
from jax.experimental.pallas import tpu as pltpu
from jax.experimental.pallas import tpu_sc as plsc

N = 10000
D = 128
E = 320000

NC = 2
NS = 16
NPAD = 10240
RPT = NPAD // NS
EPT = E // (NC * NS)
EB = 80
NCHUNK = EPT // EB
ZR = 64
DW = 16

RB = 2048
TC_GRID = NPAD // RB

_sc_mesh = plsc.VectorSubcoreMesh(
    core_axis_name="c", subcore_axis_name="s", num_cores=NC, num_subcores=NS)


def _zero_fill(buf, rows):
  cols = buf.shape[1] // 16

  def body(i, carry):
    buf[i // cols, pl.ds((i % cols) * 16, 16)] = jnp.zeros((16,), jnp.float32)
    return carry

  lax.fori_loop(0, rows * cols, body, 0)


def _sc_deg_body(col_hbm, out0, out1, onev, colv, acc, sem):
  c = lax.axis_index("c")
  s = lax.axis_index("s")
  cols = DW // 16
  _zero_fill(onev, EB)

  def zero_slice(i, carry):
    pltpu.sync_copy(onev, acc.at[pl.ds(s * RPT + i * EB, EB)])
    return carry

  lax.fori_loop(0, RPT // EB, zero_slice, 0)

  def fill(i, carry):
    onev[i // cols, pl.ds((i % cols) * 16, 16)] = jnp.ones((16,), jnp.float32)
    return carry

  lax.fori_loop(0, EB * cols, fill, 0)
  plsc.subcore_barrier()

  base = (c * NS + s) * EPT

  def chunk(k, carry):
    pltpu.sync_copy(col_hbm.at[pl.ds(base + k * EB, EB)], colv)
    pltpu.sync_copy(onev, acc.at[colv], add=True)
    return carry

  lax.fori_loop(0, NCHUNK, chunk, 0)
  plsc.subcore_barrier()

  r0 = s * RPT

  @pl.when(c == 0)
  def _():
    pltpu.sync_copy(acc.at[pl.ds(r0, RPT)], out0.at[pl.ds(r0, RPT)])

  @pl.when(c == 1)
  def _():
    pltpu.sync_copy(acc.at[pl.ds(r0, RPT)], out1.at[pl.ds(r0, RPT)])


_deg_call = pl.kernel(
    _sc_deg_body,
    out_type=[
        jax.ShapeDtypeStruct((NPAD, DW), jnp.float32),
        jax.ShapeDtypeStruct((NPAD, DW), jnp.float32),
    ],
    mesh=_sc_mesh,
    scratch_types=[
        pltpu.VMEM((EB, DW), jnp.float32),
        pltpu.VMEM((EB,), jnp.int32),
        pltpu.VMEM_SHARED((NPAD, DW), jnp.float32),
        pltpu.SemaphoreType.DMA,
    ],
)


def _sc_agg_body(hp_hbm, row_hbm, col_hbm, out0, out1,
                 zbuf, rowv, colv, gatv, acc, sem):
  c = lax.axis_index("c")
  s = lax.axis_index("s")
  _zero_fill(zbuf, ZR)

  def zero_slice(i, carry):
    pltpu.sync_copy(zbuf, acc.at[pl.ds(s * RPT + i * ZR, ZR)])
    return carry

  lax.fori_loop(0, RPT // ZR, zero_slice, 0)
  plsc.subcore_barrier()

  base = (c * NS + s) * EPT

  def chunk(k, carry):
    off = base + k * EB
    pltpu.sync_copy(row_hbm.at[pl.ds(off, EB)], rowv)
    pltpu.sync_copy(col_hbm.at[pl.ds(off, EB)], colv)
    pltpu.async_copy(hp_hbm.at[rowv], gatv, sem).wait()
    pltpu.sync_copy(gatv, acc.at[colv], add=True)
    return carry

  lax.fori_loop(0, NCHUNK, chunk, 0)
  plsc.subcore_barrier()

  r0 = s * RPT

  @pl.when(c == 0)
  def _():
    pltpu.sync_copy(acc.at[pl.ds(r0, RPT)], out0.at[pl.ds(r0, RPT)])

  @pl.when(c == 1)
  def _():
    pltpu.sync_copy(acc.at[pl.ds(r0, RPT)], out1.at[pl.ds(r0, RPT)])


_agg_call = pl.kernel(
    _sc_agg_body,
    out_type=[
        jax.ShapeDtypeStruct((NPAD, D), jnp.float32),
        jax.ShapeDtypeStruct((NPAD, D), jnp.float32),
    ],
    mesh=_sc_mesh,
    scratch_types=[
        pltpu.VMEM((ZR, D), jnp.float32),
        pltpu.VMEM((EB,), jnp.int32),
        pltpu.VMEM((EB,), jnp.int32),
        pltpu.VMEM((EB, D), jnp.float32),
        pltpu.VMEM_SHARED((NPAD, D), jnp.float32),
        pltpu.SemaphoreType.DMA,
    ],
)


_row_spec = pl.BlockSpec((RB, D), lambda i: (i, 0))
_dis_spec = pl.BlockSpec((RB, 1), lambda i: (i, 0))
_w_spec = pl.BlockSpec((D, D), lambda i: (0, 0))
_b_spec = pl.BlockSpec((1, D), lambda i: (0, 0))


def _tc0_body(x_ref, d0_ref, d1_ref, w_ref, dis_ref, p_ref):
  deg = d0_ref[...] + d1_ref[...] + 1.0
  dis = lax.rsqrt(deg)
  dis_ref[...] = dis
  p_ref[...] = jnp.dot(x_ref[...], w_ref[...],
                       preferred_element_type=jnp.float32) * dis


_tc0 = pl.pallas_call(
    _tc0_body,
    grid=(TC_GRID,),
    in_specs=[_row_spec, _dis_spec, _dis_spec, _w_spec],
    out_specs=[_dis_spec, _row_spec],
    out_shape=[
        jax.ShapeDtypeStruct((NPAD, 1), jnp.float32),
        jax.ShapeDtypeStruct((NPAD, D), jnp.float32),
    ],
)


def _tc_mid_body(emit_y, res, a0_ref, a1_ref, p_ref, dis_ref, b_ref, w_ref,
                 *out_refs):
  if res:
    hres_ref = w_ref
    w_ref = out_refs[0]
    out_refs = out_refs[1:]
  dis = dis_ref[...]
  y = dis * (a0_ref[...] + a1_ref[...] + p_ref[...]) + b_ref[...]
  y = jnp.maximum(y, 0.0)
  if res:
    y = (hres_ref[...] + y) * 0.5
  if emit_y:
    out_refs[0][...] = y
    pout = out_refs[1]
  else:
    pout = out_refs[0]
  pout[...] = jnp.dot(y, w_ref[...], preferred_element_type=jnp.float32) * dis


def _make_tc_mid(emit_y, res):
  in_specs = [_row_spec, _row_spec, _row_spec, _dis_spec, _b_spec]
  if res:
    in_specs.append(_row_spec)
  in_specs.append(_w_spec)
  out_specs = []
  out_shape = []
  if emit_y:
    out_specs.append(_row_spec)
    out_shape.append(jax.ShapeDtypeStruct((NPAD, D), jnp.float32))
  out_specs.append(_row_spec)
  out_shape.append(jax.ShapeDtypeStruct((NPAD, D), jnp.float32))

  def body(a0, a1, p, dis, b, *rest):
    if res:
      hres, w = rest[0], rest[1]
      outs = rest[2:]
      _tc_mid_body(emit_y, True, a0, a1, p, dis, b, hres, w, *outs)
    else:
      w = rest[0]
      outs = rest[1:]
      _tc_mid_body(emit_y, False, a0, a1, p, dis, b, w, *outs)

  call = pl.pallas_call(
      body,
      grid=(TC_GRID,),
      in_specs=in_specs,
      out_specs=out_specs,
      out_shape=out_shape,
  )
  if emit_y:
    return call
  return lambda *args: call(*args)[0]


_tc_mid_plain = _make_tc_mid(emit_y=False, res=False)
_tc_mid_emit = _make_tc_mid(emit_y=True, res=False)
_tc_mid_res = _make_tc_mid(emit_y=True, res=True)


def _tc_final_body(a0_ref, a1_ref, p_ref, dis_ref, b_ref, o_ref):
  o_ref[...] = (dis_ref[...] * (a0_ref[...] + a1_ref[...] + p_ref[...])
                + b_ref[...])


_tc_final = pl.pallas_call(
    _tc_final_body,
    grid=(TC_GRID,),
    in_specs=[_row_spec, _row_spec, _row_spec, _dis_spec, _b_spec],
    out_specs=_row_spec,
    out_shape=jax.ShapeDtypeStruct((NPAD, D), jnp.float32),
)


def kernel(x, edge_index, weights, biases):
  row = edge_index[0]
  col = edge_index[1]
  xp = jnp.pad(x, ((0, NPAD - N), (0, 0)))
  b2 = [b.reshape(1, D) for b in biases]

  deg0, deg1 = _deg_call(col)
  dis, p = _tc0(xp, deg0[:, :1], deg1[:, :1], weights[0])

  a0, a1 = _agg_call(p, row, col)
  h, p = _tc_mid_emit(a0, a1, p, dis, b2[0], weights[1])

  wi = 1
  for _ in range(3):
    a0, a1 = _agg_call(p, row, col)
    p = _tc_mid_plain(a0, a1, p, dis, b2[wi], weights[wi + 1])
    a0, a1 = _agg_call(p, row, col)
    h, p = _tc_mid_res(a0, a1, p, dis, b2[wi + 1], h, weights[wi + 2])
    wi += 2

  a0, a1 = _agg_call(p, row, col)
  out = _tc_final(a0, a1, p, dis, b2[wi])
  return (out[:N], h[:N])
